# Initial kernel scaffold; baseline (speedup 1.0000x reference)
import jax
import jax.numpy as jnp
from jax.experimental import pallas as pl


def kernel(x, edge_index, edge_attr, batch, Wf1, bf1, Ws1, bs1, Wf2, bf2, Ws2, bs2, Wf3, bf3, Ws3, bs3, W1, b1, W2, b2, W3, b3):
    raise NotImplementedError("write your pallas kernel here")


# speedup vs baseline: 3.6626x; 3.6626x over previous
import functools

import jax
import jax.numpy as jnp
from jax import lax
from jax.experimental import pallas as pl
from jax.experimental.pallas import tpu as pltpu
from jax.experimental.pallas import tpu_sc as plsc

F = 9
DE = 4
NG = 64
NC = 2
NS = 16
CHUNK = 128
BLKN = 512
BLKE = 2048
PW = 24
MW = 16

_LOG_POLY = (
    0.030102625011658456,
    -0.2806325404494927,
    1.1048082361987304,
    -2.4208125632180866,
    3.4982279012091095,
    -1.9316715417207186,
)


def _sigmoid(v):
    return 1.0 / (1.0 + jnp.exp(-v))


def _softplus(v):
    t = 1.0 + jnp.exp(-jnp.abs(v))
    p = jnp.float32(_LOG_POLY[0])
    for c in _LOG_POLY[1:]:
        p = p * t + jnp.float32(c)
    return jnp.maximum(v, 0.0) + p


def _sc_edge_stage(pd, ps, eaw, dstp, srcp, zrows, *, nacc, rpt, epw, cpw):
    mesh = plsc.VectorSubcoreMesh(core_axis_name="c", subcore_axis_name="s")

    def body(pd_h, ps_h, ea_h, dst_h, src_h, zr_h, out_h,
             dstb, srcb, gd, gs, eab, mb, accum, sem_gd, sem_gs):
        c = lax.axis_index("c")
        s = lax.axis_index("s")
        w = c * NS + s

        pltpu.sync_copy(zr_h, accum.at[pl.ds(s * rpt, rpt), :])
        plsc.subcore_barrier()

        i16 = lax.iota(jnp.int32, 16)
        cols = [jnp.full((16,), j, jnp.int32) for j in range(2 * F)]

        def chunk_body(g, carry):
            base = w * epw + g * CHUNK
            pltpu.sync_copy(dst_h.at[pl.ds(base, CHUNK)], dstb)
            pltpu.sync_copy(src_h.at[pl.ds(base, CHUNK)], srcb)
            cp_d = pltpu.async_copy(pd_h.at[dstb], gd, sem_gd)
            cp_s = pltpu.async_copy(ps_h.at[srcb], gs, sem_gs)
            pltpu.sync_copy(ea_h.at[pl.ds(base, CHUNK), :], eab)
            cp_d.wait()
            cp_s.wait()
            for grp in range(CHUNK // 16):
                rows = i16 + (grp * 16)
                dv = [plsc.load_gather(gd, [rows, cols[j]]) for j in range(2 * F)]
                sv = [plsc.load_gather(gs, [rows, cols[j]]) for j in range(2 * F)]
                ev = [plsc.load_gather(eab, [rows, cols[j]]) for j in range(2 * F)]
                for j in range(F):
                    fl = dv[j] + sv[j] + ev[j]
                    sl = dv[F + j] + sv[F + j] + ev[F + j]
                    m = _sigmoid(fl) * _softplus(sl)
                    plsc.store_scatter(mb, [rows, cols[j]], m)
            pltpu.sync_copy(mb, accum.at[dstb], add=True)
            return carry

        lax.fori_loop(0, cpw, chunk_body, 0)
        plsc.subcore_barrier()
        pltpu.sync_copy(accum.at[pl.ds(s * rpt, rpt), :],
                        out_h.at[pl.ds(c * nacc + s * rpt, rpt), :])

    run = pl.kernel(
        body,
        out_type=jax.ShapeDtypeStruct((2 * nacc, MW), jnp.float32),
        mesh=mesh,
        scratch_types=[
            pltpu.VMEM((CHUNK,), jnp.int32),
            pltpu.VMEM((CHUNK,), jnp.int32),
            pltpu.VMEM((CHUNK, PW), jnp.float32),
            pltpu.VMEM((CHUNK, PW), jnp.float32),
            pltpu.VMEM((CHUNK, PW), jnp.float32),
            pltpu.VMEM((CHUNK, MW), jnp.float32),
            pltpu.VMEM_SHARED((nacc, MW), jnp.float32),
            pltpu.SemaphoreType.DMA,
            pltpu.SemaphoreType.DMA,
        ],
        compiler_params=pltpu.CompilerParams(use_tc_tiling_on_sc=False,
                                             needs_layout_passes=False),
    )
    return run(pd, ps, eaw, dstp, srcp, zrows)


def _proj_kernel(h, wcat, *, n, nacc):
    nblk = pl.cdiv(nacc, BLKN)

    def body(h_ref, w_ref, pd_ref, ps_ref):
        p = jnp.dot(h_ref[...], w_ref[...], preferred_element_type=jnp.float32)
        zpad = jnp.zeros((BLKN, PW - 2 * F), jnp.float32)
        pd_ref[...] = jnp.concatenate([p[:, : 2 * F], zpad], axis=1)
        ps_ref[...] = jnp.concatenate([p[:, 2 * F :], zpad], axis=1)

    return pl.pallas_call(
        body,
        grid=(nblk,),
        in_specs=[
            pl.BlockSpec((BLKN, F), lambda i: (i, 0)),
            pl.BlockSpec((F, 4 * F), lambda i: (0, 0)),
        ],
        out_specs=[
            pl.BlockSpec((BLKN, PW), lambda i: (i, 0)),
            pl.BlockSpec((BLKN, PW), lambda i: (i, 0)),
        ],
        out_shape=[
            jax.ShapeDtypeStruct((nacc, PW), jnp.float32),
            jax.ShapeDtypeStruct((nacc, PW), jnp.float32),
        ],
    )(h, wcat)


def _ea_kernel(eap, wea_all, bias_all, *, ep):
    nblk = ep // BLKE

    def body(ea_ref, w_ref, b_ref, o1, o2, o3):
        p = jnp.dot(ea_ref[...], w_ref[...], preferred_element_type=jnp.float32)
        p = p + b_ref[...]
        zpad = jnp.zeros((BLKE, PW - 2 * F), jnp.float32)
        o1[...] = jnp.concatenate([p[:, 0 : 2 * F], zpad], axis=1)
        o2[...] = jnp.concatenate([p[:, 2 * F : 4 * F], zpad], axis=1)
        o3[...] = jnp.concatenate([p[:, 4 * F : 6 * F], zpad], axis=1)

    return pl.pallas_call(
        body,
        grid=(nblk,),
        in_specs=[
            pl.BlockSpec((BLKE, DE), lambda i: (i, 0)),
            pl.BlockSpec((DE, 6 * F), lambda i: (0, 0)),
            pl.BlockSpec((1, 6 * F), lambda i: (0, 0)),
        ],
        out_specs=[pl.BlockSpec((BLKE, PW), lambda i: (i, 0))] * 3,
        out_shape=[jax.ShapeDtypeStruct((ep, PW), jnp.float32)] * 3,
    )(eap, wea_all, bias_all)


def _pool_block(i, hn, b_ref, pool_ref, *, n):
    gids = lax.broadcasted_iota(jnp.int32, (BLKN, NG), 1)
    rows = lax.broadcasted_iota(jnp.int32, (BLKN, 1), 0) + i * BLKN
    onehot = (b_ref[...] == gids) & (rows < n)
    oh = onehot.astype(jnp.float32)
    sums = lax.dot_general(hn, oh, (((0,), (0,)), ((), ())),
                           preferred_element_type=jnp.float32)
    cnt = jnp.sum(oh, axis=0, keepdims=True)
    mns, mxs = [], []
    for j in range(F):
        hj = hn[:, j : j + 1]
        mns.append(jnp.min(jnp.where(onehot, hj, jnp.inf), axis=0, keepdims=True))
        mxs.append(jnp.max(jnp.where(onehot, hj, -jnp.inf), axis=0, keepdims=True))
    mn = jnp.concatenate(mns, axis=0)
    mx = jnp.concatenate(mxs, axis=0)

    @pl.when(i == 0)
    def _():
        pool_ref[...] = jnp.concatenate(
            [jnp.full((F, NG), jnp.inf, jnp.float32),
             jnp.full((F, NG), -jnp.inf, jnp.float32),
             jnp.zeros((F + 1, NG), jnp.float32)], axis=0)

    cur = pool_ref[...]
    pool_ref[...] = jnp.concatenate(
        [jnp.minimum(cur[0:F], mn),
         jnp.maximum(cur[F : 2 * F], mx),
         cur[2 * F : 3 * F] + sums,
         cur[3 * F : 3 * F + 1] + cnt], axis=0)


def _upd_kernel(h, a0, a1, batch2, wnext, *, n, nacc):
    nblk = pl.cdiv(nacc, BLKN)

    def body(h_ref, a0_ref, a1_ref, b_ref, w_ref, h_out, pd_ref, ps_ref, pool_ref):
        i = pl.program_id(0)
        hn = jnp.maximum(h_ref[...] + a0_ref[...] + a1_ref[...], 0.0)
        h_out[...] = hn
        p = jnp.dot(hn, w_ref[...], preferred_element_type=jnp.float32)
        zpad = jnp.zeros((BLKN, PW - 2 * F), jnp.float32)
        pd_ref[...] = jnp.concatenate([p[:, : 2 * F], zpad], axis=1)
        ps_ref[...] = jnp.concatenate([p[:, 2 * F :], zpad], axis=1)
        _pool_block(i, hn, b_ref, pool_ref, n=n)

    return pl.pallas_call(
        body,
        grid=(nblk,),
        in_specs=[
            pl.BlockSpec((BLKN, F), lambda i: (i, 0)),
            pl.BlockSpec((BLKN, F), lambda i: (i, 0)),
            pl.BlockSpec((BLKN, F), lambda i: (i, 0)),
            pl.BlockSpec((BLKN, 1), lambda i: (i, 0)),
            pl.BlockSpec((F, 4 * F), lambda i: (0, 0)),
        ],
        out_specs=[
            pl.BlockSpec((BLKN, F), lambda i: (i, 0)),
            pl.BlockSpec((BLKN, PW), lambda i: (i, 0)),
            pl.BlockSpec((BLKN, PW), lambda i: (i, 0)),
            pl.BlockSpec((3 * F + 1, NG), lambda i: (0, 0)),
        ],
        out_shape=[
            jax.ShapeDtypeStruct((n, F), jnp.float32),
            jax.ShapeDtypeStruct((nacc, PW), jnp.float32),
            jax.ShapeDtypeStruct((nacc, PW), jnp.float32),
            jax.ShapeDtypeStruct((3 * F + 1, NG), jnp.float32),
        ],
    )(h, a0, a1, batch2, wnext)


def _upd_last_kernel(h, a0, a1, batch2, *, n):
    nblk = pl.cdiv(n, BLKN)

    def body(h_ref, a0_ref, a1_ref, b_ref, pool_ref):
        i = pl.program_id(0)
        hn = jnp.maximum(h_ref[...] + a0_ref[...] + a1_ref[...], 0.0)
        _pool_block(i, hn, b_ref, pool_ref, n=n)

    return pl.pallas_call(
        body,
        grid=(nblk,),
        in_specs=[
            pl.BlockSpec((BLKN, F), lambda i: (i, 0)),
            pl.BlockSpec((BLKN, F), lambda i: (i, 0)),
            pl.BlockSpec((BLKN, F), lambda i: (i, 0)),
            pl.BlockSpec((BLKN, 1), lambda i: (i, 0)),
        ],
        out_specs=[pl.BlockSpec((3 * F + 1, NG), lambda i: (0, 0))],
        out_shape=[jax.ShapeDtypeStruct((3 * F + 1, NG), jnp.float32)],
    )(h, a0, a1, batch2)


def _head_kernel(p1, p2, p3, w1t, b1r, w2t, b2r, w3t, b3r, *, num_out):
    def body(p1_ref, p2_ref, p3_ref, w1_ref, b1_ref, w2_ref, b2_ref,
             w3_ref, b3_ref, ot_ref, gt_ref):
        def finish(pref):
            pv = pref[...]
            mean = pv[2 * F : 3 * F] / jnp.maximum(pv[3 * F : 3 * F + 1], 1.0)
            return jnp.concatenate([pv[0 : 2 * F], mean], axis=0)

        g = finish(p1_ref) + finish(p2_ref) + finish(p3_ref)
        gt_ref[...] = g
        o = jnp.dot(w1_ref[...], g, preferred_element_type=jnp.float32)
        o = jnp.maximum(o + b1_ref[...], 0.0)
        o = jnp.dot(w2_ref[...], o, preferred_element_type=jnp.float32)
        o = jnp.maximum(o + b2_ref[...], 0.0)
        o = jnp.dot(w3_ref[...], o, preferred_element_type=jnp.float32)
        ot_ref[...] = o + b3_ref[...]

    return pl.pallas_call(
        body,
        out_shape=[
            jax.ShapeDtypeStruct((num_out, NG), jnp.float32),
            jax.ShapeDtypeStruct((3 * F, NG), jnp.float32),
        ],
    )(p1, p2, p3, w1t, b1r, w2t, b2r, w3t, b3r)


def _pack_layer(Wf, bf, Ws, bs):
    wd = jnp.concatenate([Wf[0:F], Ws[0:F]], axis=1)
    wsrc = jnp.concatenate([Wf[F : 2 * F], Ws[F : 2 * F]], axis=1)
    wea = jnp.concatenate([Wf[2 * F :], Ws[2 * F :]], axis=1)
    bia = jnp.concatenate([bf, bs])[None, :]
    return jnp.concatenate([wd, wsrc], axis=1), wea, bia


def kernel(x, edge_index, edge_attr, batch,
           Wf1, bf1, Ws1, bs1, Wf2, bf2, Ws2, bs2, Wf3, bf3, Ws3, bs3,
           W1, b1, W2, b2, W3, b3):
    n = x.shape[0]
    e = edge_index.shape[1]
    num_out = W3.shape[1]

    rpt = n // NS + 6
    nacc = NS * rpt
    nw = NC * NS
    cpw = -(-e // (nw * CHUNK))
    cpw = cpw + (cpw % 2)
    epw = cpw * CHUNK
    ep = nw * epw

    pad = ep - e
    padi = jnp.arange(pad, dtype=jnp.int32)
    dstp = jnp.concatenate([edge_index[1], n + (padi % (nacc - n))])
    srcp = jnp.concatenate([edge_index[0], (padi * 37) % n])
    eap = jnp.concatenate([edge_attr, jnp.zeros((pad, DE), jnp.float32)])
    zrows = jnp.zeros((rpt, MW), jnp.float32)
    batch2 = batch.reshape(n, 1)

    wc1, wea1, bia1 = _pack_layer(Wf1, bf1, Ws1, bs1)
    wc2, wea2, bia2 = _pack_layer(Wf2, bf2, Ws2, bs2)
    wc3, wea3, bia3 = _pack_layer(Wf3, bf3, Ws3, bs3)
    wea_all = jnp.concatenate([wea1, wea2, wea3], axis=1)
    bia_all = jnp.concatenate([bia1, bia2, bia3], axis=1)

    ea1, ea2, ea3 = _ea_kernel(eap, wea_all, bia_all, ep=ep)
    eas = (ea1, ea2, ea3)
    wnext = (None, wc2, wc3)

    pd, ps = _proj_kernel(x, wc1, n=n, nacc=nacc)
    h = x
    pools = []
    for l in range(3):
        accf = _sc_edge_stage(pd, ps, eas[l], dstp, srcp, zrows,
                              nacc=nacc, rpt=rpt, epw=epw, cpw=cpw)
        a0 = accf[:n, :F]
        a1 = accf[nacc : nacc + n, :F]
        if l < 2:
            h, pd, ps, pool = _upd_kernel(h, a0, a1, batch2, wnext[l + 1],
                                          n=n, nacc=nacc)
        else:
            (pool,) = _upd_last_kernel(h, a0, a1, batch2, n=n)
        pools.append(pool)

    ot, gt = _head_kernel(pools[0], pools[1], pools[2],
                          W1.T, b1.reshape(-1, 1),
                          W2.T, b2.reshape(-1, 1),
                          W3.T, b3.reshape(-1, 1), num_out=num_out)
    o = ot.T
    encoding = gt.T
    return (o, lax.stop_gradient(encoding))


# speedup vs baseline: 3.8229x; 1.0438x over previous
import functools

import jax
import jax.numpy as jnp
from jax import lax
from jax.experimental import pallas as pl
from jax.experimental.pallas import tpu as pltpu
from jax.experimental.pallas import tpu_sc as plsc

F = 9
DE = 4
NG = 64
NC = 2
NS = 16
CH = 128
BLKN = 512
BLKE = 2048
PW = 24
MW = 16
EAW = 128

_LOG_POLY = (
    0.030102625011658456,
    -0.2806325404494927,
    1.1048082361987304,
    -2.4208125632180866,
    3.4982279012091095,
    -1.9316715417207186,
)


def _sigmoid(v):
    return 1.0 / (1.0 + jnp.exp(-v))


def _softplus(v):
    t = 1.0 + jnp.exp(-jnp.abs(v))
    p = jnp.float32(_LOG_POLY[0])
    for c in _LOG_POLY[1:]:
        p = p * t + jnp.float32(c)
    return jnp.maximum(v, 0.0) + p


def _sc_edge_stage(pd, ps, eaw, dstp, srcp, zrows, *, nacc, rpt, epw, cpw):
    mesh = plsc.VectorSubcoreMesh(core_axis_name="c", subcore_axis_name="s")

    def body(pd_h, ps_h, ea_h, dst_h, src_h, zr_h, out_h,
             dstb, srcb, gd, gs, eab, mb, accum, sem_gd, sem_gs):
        c = lax.axis_index("c")
        s = lax.axis_index("s")
        w = c * NS + s

        pltpu.sync_copy(zr_h, accum.at[pl.ds(s * rpt, rpt), :])
        plsc.subcore_barrier()

        i16 = lax.iota(jnp.int32, 16)
        cols = [jnp.full((16,), j, jnp.int32) for j in range(2 * F)]

        def chunk_body(g, carry):
            base = w * epw + g * CH
            pltpu.sync_copy(dst_h.at[pl.ds(base, CH)], dstb)
            pltpu.sync_copy(src_h.at[pl.ds(base, CH)], srcb)
            cp_d = pltpu.async_copy(pd_h.at[dstb], gd, sem_gd)
            cp_s = pltpu.async_copy(ps_h.at[srcb], gs, sem_gs)
            pltpu.sync_copy(ea_h.at[pl.ds(base // 4, CH // 4), :], eab)
            cp_d.wait()
            cp_s.wait()
            for grp in range(CH // 16):
                rows = i16 + (grp * 16)
                erow = lax.shift_right_logical(rows, 2)
                ecol = lax.shift_left(rows & 3, 5)
                dv = [plsc.load_gather(gd, [rows, cols[j]])
                      for j in range(2 * F)]
                sv = [plsc.load_gather(gs, [rows, cols[j]])
                      for j in range(2 * F)]
                ev = [plsc.load_gather(eab, [erow, ecol + j])
                      for j in range(2 * F)]
                for j in range(F):
                    fl = dv[j] + sv[j] + ev[j]
                    sl = dv[F + j] + sv[F + j] + ev[F + j]
                    m = _sigmoid(fl) * _softplus(sl)
                    plsc.store_scatter(mb, [rows, cols[j]], m)
            pltpu.sync_copy(mb, accum.at[dstb], add=True)
            return carry

        lax.fori_loop(0, cpw, chunk_body, 0)
        plsc.subcore_barrier()
        pltpu.sync_copy(accum.at[pl.ds(s * rpt, rpt), :],
                        out_h.at[pl.ds(c * nacc + s * rpt, rpt), :])

    run = pl.kernel(
        body,
        out_type=jax.ShapeDtypeStruct((2 * nacc, MW), jnp.float32),
        mesh=mesh,
        scratch_types=[
            pltpu.VMEM((CH,), jnp.int32),
            pltpu.VMEM((CH,), jnp.int32),
            pltpu.VMEM((CH, PW), jnp.float32),
            pltpu.VMEM((CH, PW), jnp.float32),
            pltpu.VMEM((CH // 4, 128), jnp.float32),
            pltpu.VMEM((CH, MW), jnp.float32),
            pltpu.VMEM_SHARED((nacc, MW), jnp.float32),
            pltpu.SemaphoreType.DMA,
            pltpu.SemaphoreType.DMA,
        ],
        compiler_params=pltpu.CompilerParams(use_tc_tiling_on_sc=False,
                                             needs_layout_passes=False),
    )
    return run(pd, ps, eaw, dstp, srcp, zrows)


def _proj_kernel(h, wcat, *, n, nacc):
    nblk = pl.cdiv(nacc, BLKN)

    def body(h_ref, w_ref, pd_ref, ps_ref):
        p = jnp.dot(h_ref[...], w_ref[...], preferred_element_type=jnp.float32)
        zpad = jnp.zeros((BLKN, PW - 2 * F), jnp.float32)
        pd_ref[...] = jnp.concatenate([p[:, : 2 * F], zpad], axis=1)
        ps_ref[...] = jnp.concatenate([p[:, 2 * F :], zpad], axis=1)

    return pl.pallas_call(
        body,
        grid=(nblk,),
        in_specs=[
            pl.BlockSpec((BLKN, F), lambda i: (i, 0)),
            pl.BlockSpec((F, 4 * F), lambda i: (0, 0)),
        ],
        out_specs=[
            pl.BlockSpec((BLKN, PW), lambda i: (i, 0)),
            pl.BlockSpec((BLKN, PW), lambda i: (i, 0)),
        ],
        out_shape=[
            jax.ShapeDtypeStruct((nacc, PW), jnp.float32),
            jax.ShapeDtypeStruct((nacc, PW), jnp.float32),
        ],
    )(h, wcat)


def _ea_kernel(ea4, wbig, bbig, *, ep):
    blk = BLKE // 4
    nblk = pl.cdiv(ea4.shape[0], blk)

    def body(ea_ref, w_ref, b_ref, o1, o2, o3):
        for l, o_ref in enumerate((o1, o2, o3)):
            p = jnp.dot(ea_ref[...], w_ref[l], preferred_element_type=jnp.float32)
            o_ref[...] = p + b_ref[l]

    return pl.pallas_call(
        body,
        grid=(nblk,),
        in_specs=[
            pl.BlockSpec((blk, 4 * DE), lambda i: (i, 0)),
            pl.BlockSpec((3, 4 * DE, 128), lambda i: (0, 0, 0)),
            pl.BlockSpec((3, 1, 128), lambda i: (0, 0, 0)),
        ],
        out_specs=[pl.BlockSpec((blk, 128), lambda i: (i, 0))] * 3,
        out_shape=[jax.ShapeDtypeStruct((ep // 4, 128), jnp.float32)] * 3,
    )(ea4, wbig, bbig)


def _pool_block(i, hn, b_ref, pool_ref, *, n):
    gids = lax.broadcasted_iota(jnp.int32, (BLKN, NG), 1)
    rows = lax.broadcasted_iota(jnp.int32, (BLKN, 1), 0) + i * BLKN
    onehot = (b_ref[...] == gids) & (rows < n)
    oh = onehot.astype(jnp.float32)
    sums = lax.dot_general(hn, oh, (((0,), (0,)), ((), ())),
                           preferred_element_type=jnp.float32)
    cnt = jnp.sum(oh, axis=0, keepdims=True)
    mns, mxs = [], []
    for j in range(F):
        hj = hn[:, j : j + 1]
        mns.append(jnp.min(jnp.where(onehot, hj, jnp.inf), axis=0, keepdims=True))
        mxs.append(jnp.max(jnp.where(onehot, hj, -jnp.inf), axis=0, keepdims=True))
    mn = jnp.concatenate(mns, axis=0)
    mx = jnp.concatenate(mxs, axis=0)

    @pl.when(i == 0)
    def _():
        pool_ref[...] = jnp.concatenate(
            [jnp.full((F, NG), jnp.inf, jnp.float32),
             jnp.full((F, NG), -jnp.inf, jnp.float32),
             jnp.zeros((F + 1, NG), jnp.float32)], axis=0)

    cur = pool_ref[...]
    pool_ref[...] = jnp.concatenate(
        [jnp.minimum(cur[0:F], mn),
         jnp.maximum(cur[F : 2 * F], mx),
         cur[2 * F : 3 * F] + sums,
         cur[3 * F : 3 * F + 1] + cnt], axis=0)


def _upd_kernel(h, a0, a1, batch2, wnext, *, n, nacc):
    nblk = pl.cdiv(nacc, BLKN)

    def body(h_ref, a0_ref, a1_ref, b_ref, w_ref, h_out, pd_ref, ps_ref, pool_ref):
        i = pl.program_id(0)
        hn = jnp.maximum(h_ref[...] + a0_ref[...] + a1_ref[...], 0.0)
        h_out[...] = hn
        p = jnp.dot(hn, w_ref[...], preferred_element_type=jnp.float32)
        zpad = jnp.zeros((BLKN, PW - 2 * F), jnp.float32)
        pd_ref[...] = jnp.concatenate([p[:, : 2 * F], zpad], axis=1)
        ps_ref[...] = jnp.concatenate([p[:, 2 * F :], zpad], axis=1)
        _pool_block(i, hn, b_ref, pool_ref, n=n)

    return pl.pallas_call(
        body,
        grid=(nblk,),
        in_specs=[
            pl.BlockSpec((BLKN, F), lambda i: (i, 0)),
            pl.BlockSpec((BLKN, F), lambda i: (i, 0)),
            pl.BlockSpec((BLKN, F), lambda i: (i, 0)),
            pl.BlockSpec((BLKN, 1), lambda i: (i, 0)),
            pl.BlockSpec((F, 4 * F), lambda i: (0, 0)),
        ],
        out_specs=[
            pl.BlockSpec((BLKN, F), lambda i: (i, 0)),
            pl.BlockSpec((BLKN, PW), lambda i: (i, 0)),
            pl.BlockSpec((BLKN, PW), lambda i: (i, 0)),
            pl.BlockSpec((3 * F + 1, NG), lambda i: (0, 0)),
        ],
        out_shape=[
            jax.ShapeDtypeStruct((n, F), jnp.float32),
            jax.ShapeDtypeStruct((nacc, PW), jnp.float32),
            jax.ShapeDtypeStruct((nacc, PW), jnp.float32),
            jax.ShapeDtypeStruct((3 * F + 1, NG), jnp.float32),
        ],
    )(h, a0, a1, batch2, wnext)


def _upd_last_kernel(h, a0, a1, batch2, *, n):
    nblk = pl.cdiv(n, BLKN)

    def body(h_ref, a0_ref, a1_ref, b_ref, pool_ref):
        i = pl.program_id(0)
        hn = jnp.maximum(h_ref[...] + a0_ref[...] + a1_ref[...], 0.0)
        _pool_block(i, hn, b_ref, pool_ref, n=n)

    return pl.pallas_call(
        body,
        grid=(nblk,),
        in_specs=[
            pl.BlockSpec((BLKN, F), lambda i: (i, 0)),
            pl.BlockSpec((BLKN, F), lambda i: (i, 0)),
            pl.BlockSpec((BLKN, F), lambda i: (i, 0)),
            pl.BlockSpec((BLKN, 1), lambda i: (i, 0)),
        ],
        out_specs=[pl.BlockSpec((3 * F + 1, NG), lambda i: (0, 0))],
        out_shape=[jax.ShapeDtypeStruct((3 * F + 1, NG), jnp.float32)],
    )(h, a0, a1, batch2)


def _head_kernel(p1, p2, p3, w1t, b1r, w2t, b2r, w3t, b3r, *, num_out):
    def body(p1_ref, p2_ref, p3_ref, w1_ref, b1_ref, w2_ref, b2_ref,
             w3_ref, b3_ref, ot_ref, gt_ref):
        def finish(pref):
            pv = pref[...]
            mean = pv[2 * F : 3 * F] / jnp.maximum(pv[3 * F : 3 * F + 1], 1.0)
            return jnp.concatenate([pv[0 : 2 * F], mean], axis=0)

        g = finish(p1_ref) + finish(p2_ref) + finish(p3_ref)
        gt_ref[...] = g
        o = jnp.dot(w1_ref[...], g, preferred_element_type=jnp.float32)
        o = jnp.maximum(o + b1_ref[...], 0.0)
        o = jnp.dot(w2_ref[...], o, preferred_element_type=jnp.float32)
        o = jnp.maximum(o + b2_ref[...], 0.0)
        o = jnp.dot(w3_ref[...], o, preferred_element_type=jnp.float32)
        ot_ref[...] = o + b3_ref[...]

    return pl.pallas_call(
        body,
        out_shape=[
            jax.ShapeDtypeStruct((num_out, NG), jnp.float32),
            jax.ShapeDtypeStruct((3 * F, NG), jnp.float32),
        ],
    )(p1, p2, p3, w1t, b1r, w2t, b2r, w3t, b3r)


def _pack_layer(Wf, bf, Ws, bs):
    wd = jnp.concatenate([Wf[0:F], Ws[0:F]], axis=1)
    wsrc = jnp.concatenate([Wf[F : 2 * F], Ws[F : 2 * F]], axis=1)
    wea = jnp.concatenate([Wf[2 * F :], Ws[2 * F :]], axis=1)
    bia = jnp.concatenate([bf, bs])[None, :]
    return jnp.concatenate([wd, wsrc], axis=1), wea, bia


def kernel(x, edge_index, edge_attr, batch,
           Wf1, bf1, Ws1, bs1, Wf2, bf2, Ws2, bs2, Wf3, bf3, Ws3, bs3,
           W1, b1, W2, b2, W3, b3):
    n = x.shape[0]
    e = edge_index.shape[1]
    num_out = W3.shape[1]

    rpt = n // NS + 6
    nacc = NS * rpt
    nw = NC * NS
    cpw = -(-e // (nw * CH))
    cpw = cpw + (cpw % 2)
    epw = cpw * CH
    ep = nw * epw

    pad = ep - e
    padi = jnp.arange(pad, dtype=jnp.int32)
    dstp = jnp.concatenate([edge_index[1], n + (padi % (nacc - n))])
    srcp = jnp.concatenate([edge_index[0], (padi * 37) % n])
    zrows = jnp.zeros((rpt, MW), jnp.float32)
    batch2 = batch.reshape(n, 1)

    wc1, wea1, bia1 = _pack_layer(Wf1, bf1, Ws1, bs1)
    wc2, wea2, bia2 = _pack_layer(Wf2, bf2, Ws2, bs2)
    wc3, wea3, bia3 = _pack_layer(Wf3, bf3, Ws3, bs3)
    wbig = jnp.stack([
        jnp.concatenate([
            jnp.pad(wea, ((0, 0), (32 * t, 128 - 32 * t - 2 * F)))
            for t in range(4)], axis=0)
        for wea in (wea1, wea2, wea3)])
    bbig = jnp.stack([
        sum(jnp.pad(bia, ((0, 0), (32 * t, 128 - 32 * t - 2 * F)))
            for t in range(4))
        for bia in (bia1, bia2, bia3)])
    ea4 = edge_attr.reshape(e // 4, 4 * DE)

    eas = _ea_kernel(ea4, wbig, bbig, ep=ep)
    wnext = (None, wc2, wc3)

    pd, ps = _proj_kernel(x, wc1, n=n, nacc=nacc)
    h = x
    pools = []
    for l in range(3):
        accf = _sc_edge_stage(pd, ps, eas[l], dstp, srcp, zrows,
                              nacc=nacc, rpt=rpt, epw=epw, cpw=cpw)
        a0 = accf[:n, :F]
        a1 = accf[nacc : nacc + n, :F]
        if l < 2:
            h, pd, ps, pool = _upd_kernel(h, a0, a1, batch2, wnext[l + 1],
                                          n=n, nacc=nacc)
        else:
            (pool,) = _upd_last_kernel(h, a0, a1, batch2, n=n)
        pools.append(pool)

    ot, gt = _head_kernel(pools[0], pools[1], pools[2],
                          W1.T, b1.reshape(-1, 1),
                          W2.T, b2.reshape(-1, 1),
                          W3.T, b3.reshape(-1, 1), num_out=num_out)
    o = ot.T
    encoding = gt.T
    return (o, lax.stop_gradient(encoding))


# speedup vs baseline: 5.8606x; 1.5330x over previous
import functools

import jax
import jax.numpy as jnp
from jax import lax
from jax.experimental import pallas as pl
from jax.experimental.pallas import tpu as pltpu
from jax.experimental.pallas import tpu_sc as plsc

F = 9
DE = 4
NG = 64
NC = 2
NS = 16
CH = 128
BLKN = 512
BLKE = 2048
PW = 24
MW = 16
EAW = 128

_LOG_POLY = (
    0.030102625011658456,
    -0.2806325404494927,
    1.1048082361987304,
    -2.4208125632180866,
    3.4982279012091095,
    -1.9316715417207186,
)


def _sigmoid(v):
    return 1.0 / (1.0 + jnp.exp(-v))


def _softplus(v):
    t = 1.0 + jnp.exp(-jnp.abs(v))
    p = jnp.float32(_LOG_POLY[0])
    for c in _LOG_POLY[1:]:
        p = p * t + jnp.float32(c)
    return jnp.maximum(v, 0.0) + p


def _sc_edge_stage(pd, ps, eaw, dscat, zrows, *, nacc, rpt, cpw, loff):
    mesh = plsc.VectorSubcoreMesh(core_axis_name="c", subcore_axis_name="s")

    def body(pd_h, ps_h, ea_h, dsi_h, zr_h, out_h,
             dsib, gda, gsa, eaba, mba, gdb, gsb, eabb, mbb,
             accum, sad, sas, sae, sbd, sbs, sbe):
        c = lax.axis_index("c")
        s = lax.axis_index("s")
        w = c * NS + s

        pltpu.sync_copy(zr_h, accum.at[pl.ds(s * rpt, rpt), :])
        plsc.subcore_barrier()

        i16 = lax.iota(jnp.int32, 16)
        cols = [jnp.full((16,), j, jnp.int32) for j in range(2 * F)]

        def compute(gd, gs, eab, mb):
            def grp_body(grp, carry):
                rows = i16 + grp * 16
                dv = [plsc.load_gather(gd, [rows, cols[j]])
                      for j in range(2 * F)]
                sv = [plsc.load_gather(gs, [rows, cols[j]])
                      for j in range(2 * F)]
                ev = [plsc.load_gather(eab, [rows, cols[j]])
                      for j in range(2 * F)]
                for j in range(F):
                    fl = dv[j] + sv[j] + ev[j]
                    sl = dv[F + j] + sv[F + j] + ev[F + j]
                    m = _sigmoid(fl) * _softplus(sl)
                    plsc.store_scatter(mb, [rows, cols[j]], m)
                return carry

            lax.fori_loop(0, CH // 16, grp_body, 0)

        def pair_body(i, carry):
            g0 = 2 * i
            cid = w * cpw + g0
            pltpu.sync_copy(dsi_h.at[pl.ds(cid * 2, 4), :], dsib)
            da = [pltpu.async_copy(pd_h.at[dsib.at[0]], gda, sad),
                  pltpu.async_copy(ps_h.at[dsib.at[1]], gsa, sas),
                  pltpu.async_copy(ea_h.at[pl.ds(cid * CH, CH),
                                           pl.ds(loff, PW)], eaba, sae)]
            db = [pltpu.async_copy(pd_h.at[dsib.at[2]], gdb, sbd),
                  pltpu.async_copy(ps_h.at[dsib.at[3]], gsb, sbs),
                  pltpu.async_copy(ea_h.at[pl.ds(cid * CH + CH, CH),
                                           pl.ds(loff, PW)], eabb, sbe)]
            for d in da:
                d.wait()
            compute(gda, gsa, eaba, mba)
            pltpu.sync_copy(mba, accum.at[dsib.at[0]], add=True)
            for d in db:
                d.wait()
            compute(gdb, gsb, eabb, mbb)
            pltpu.sync_copy(mbb, accum.at[dsib.at[2]], add=True)
            return carry

        lax.fori_loop(0, cpw // 2, pair_body, 0)
        plsc.subcore_barrier()
        pltpu.sync_copy(accum.at[pl.ds(s * rpt, rpt), :],
                        out_h.at[pl.ds(c * nacc + s * rpt, rpt), :])

    run = pl.kernel(
        body,
        out_type=jax.ShapeDtypeStruct((2 * nacc, MW), jnp.float32),
        mesh=mesh,
        scratch_types=(
            [pltpu.VMEM((4, 128), jnp.int32)]
            + [pltpu.VMEM((CH, PW), jnp.float32)] * 3
            + [pltpu.VMEM((CH, MW), jnp.float32)]
            + [pltpu.VMEM((CH, PW), jnp.float32)] * 3
            + [pltpu.VMEM((CH, MW), jnp.float32)]
            + [pltpu.VMEM_SHARED((nacc, MW), jnp.float32)]
            + [pltpu.SemaphoreType.DMA] * 6
        ),
        compiler_params=pltpu.CompilerParams(use_tc_tiling_on_sc=False,
                                             needs_layout_passes=False),
    )
    return run(pd, ps, eaw, dscat, zrows)


def _proj_kernel(h, wcat, *, n, nacc):
    nblk = pl.cdiv(nacc, BLKN)

    def body(h_ref, w_ref, pd_ref, ps_ref):
        p = jnp.dot(h_ref[...], w_ref[...], preferred_element_type=jnp.float32)
        zpad = jnp.zeros((BLKN, PW - 2 * F), jnp.float32)
        pd_ref[...] = jnp.concatenate([p[:, : 2 * F], zpad], axis=1)
        ps_ref[...] = jnp.concatenate([p[:, 2 * F :], zpad], axis=1)

    return pl.pallas_call(
        body,
        grid=(nblk,),
        in_specs=[
            pl.BlockSpec((BLKN, F), lambda i: (i, 0)),
            pl.BlockSpec((F, 4 * F), lambda i: (0, 0)),
        ],
        out_specs=[
            pl.BlockSpec((BLKN, PW), lambda i: (i, 0)),
            pl.BlockSpec((BLKN, PW), lambda i: (i, 0)),
        ],
        out_shape=[
            jax.ShapeDtypeStruct((nacc, PW), jnp.float32),
            jax.ShapeDtypeStruct((nacc, PW), jnp.float32),
        ],
    )(h, wcat)


def _ea_kernel(ea, wea_all, bias_all, *, ep):
    nblk = pl.cdiv(ea.shape[0], BLKE)

    def body(ea_ref, w_ref, b_ref, o_ref):
        p = jnp.dot(ea_ref[...], w_ref[...], preferred_element_type=jnp.float32)
        p = p + b_ref[...]
        zg = jnp.zeros((BLKE, 32 - 2 * F), jnp.float32)
        o_ref[...] = jnp.concatenate(
            [p[:, 0 : 2 * F], zg,
             p[:, 2 * F : 4 * F], zg,
             p[:, 4 * F : 6 * F], zg,
             jnp.zeros((BLKE, EAW - 96), jnp.float32)], axis=1)

    return pl.pallas_call(
        body,
        grid=(nblk,),
        in_specs=[
            pl.BlockSpec((BLKE, DE), lambda i: (i, 0)),
            pl.BlockSpec((DE, 6 * F), lambda i: (0, 0)),
            pl.BlockSpec((1, 6 * F), lambda i: (0, 0)),
        ],
        out_specs=[pl.BlockSpec((BLKE, EAW), lambda i: (i, 0))],
        out_shape=[jax.ShapeDtypeStruct((ep, EAW), jnp.float32)],
    )(ea, wea_all, bias_all)


def _pool_block(i, hn, b_ref, pool_ref, *, n):
    gids = lax.broadcasted_iota(jnp.int32, (BLKN, NG), 1)
    rows = lax.broadcasted_iota(jnp.int32, (BLKN, 1), 0) + i * BLKN
    onehot = (b_ref[...] == gids) & (rows < n)
    oh = onehot.astype(jnp.float32)
    sums = lax.dot_general(hn, oh, (((0,), (0,)), ((), ())),
                           preferred_element_type=jnp.float32)
    cnt = jnp.sum(oh, axis=0, keepdims=True)
    mns, mxs = [], []
    for j in range(F):
        hj = hn[:, j : j + 1]
        mns.append(jnp.min(jnp.where(onehot, hj, jnp.inf), axis=0, keepdims=True))
        mxs.append(jnp.max(jnp.where(onehot, hj, -jnp.inf), axis=0, keepdims=True))
    mn = jnp.concatenate(mns, axis=0)
    mx = jnp.concatenate(mxs, axis=0)

    @pl.when(i == 0)
    def _():
        pool_ref[...] = jnp.concatenate(
            [jnp.full((F, NG), jnp.inf, jnp.float32),
             jnp.full((F, NG), -jnp.inf, jnp.float32),
             jnp.zeros((F + 1, NG), jnp.float32)], axis=0)

    cur = pool_ref[...]
    pool_ref[...] = jnp.concatenate(
        [jnp.minimum(cur[0:F], mn),
         jnp.maximum(cur[F : 2 * F], mx),
         cur[2 * F : 3 * F] + sums,
         cur[3 * F : 3 * F + 1] + cnt], axis=0)


def _upd_kernel(h, a0, a1, batch2, wnext, *, n, nacc):
    nblk = pl.cdiv(nacc, BLKN)

    def body(h_ref, a0_ref, a1_ref, b_ref, w_ref, h_out, pd_ref, ps_ref, pool_ref):
        i = pl.program_id(0)
        hn = jnp.maximum(h_ref[...] + a0_ref[...] + a1_ref[...], 0.0)
        h_out[...] = hn
        p = jnp.dot(hn, w_ref[...], preferred_element_type=jnp.float32)
        zpad = jnp.zeros((BLKN, PW - 2 * F), jnp.float32)
        pd_ref[...] = jnp.concatenate([p[:, : 2 * F], zpad], axis=1)
        ps_ref[...] = jnp.concatenate([p[:, 2 * F :], zpad], axis=1)
        _pool_block(i, hn, b_ref, pool_ref, n=n)

    return pl.pallas_call(
        body,
        grid=(nblk,),
        in_specs=[
            pl.BlockSpec((BLKN, F), lambda i: (i, 0)),
            pl.BlockSpec((BLKN, F), lambda i: (i, 0)),
            pl.BlockSpec((BLKN, F), lambda i: (i, 0)),
            pl.BlockSpec((BLKN, 1), lambda i: (i, 0)),
            pl.BlockSpec((F, 4 * F), lambda i: (0, 0)),
        ],
        out_specs=[
            pl.BlockSpec((BLKN, F), lambda i: (i, 0)),
            pl.BlockSpec((BLKN, PW), lambda i: (i, 0)),
            pl.BlockSpec((BLKN, PW), lambda i: (i, 0)),
            pl.BlockSpec((3 * F + 1, NG), lambda i: (0, 0)),
        ],
        out_shape=[
            jax.ShapeDtypeStruct((n, F), jnp.float32),
            jax.ShapeDtypeStruct((nacc, PW), jnp.float32),
            jax.ShapeDtypeStruct((nacc, PW), jnp.float32),
            jax.ShapeDtypeStruct((3 * F + 1, NG), jnp.float32),
        ],
    )(h, a0, a1, batch2, wnext)


def _upd_last_kernel(h, a0, a1, batch2, *, n):
    nblk = pl.cdiv(n, BLKN)

    def body(h_ref, a0_ref, a1_ref, b_ref, pool_ref):
        i = pl.program_id(0)
        hn = jnp.maximum(h_ref[...] + a0_ref[...] + a1_ref[...], 0.0)
        _pool_block(i, hn, b_ref, pool_ref, n=n)

    return pl.pallas_call(
        body,
        grid=(nblk,),
        in_specs=[
            pl.BlockSpec((BLKN, F), lambda i: (i, 0)),
            pl.BlockSpec((BLKN, F), lambda i: (i, 0)),
            pl.BlockSpec((BLKN, F), lambda i: (i, 0)),
            pl.BlockSpec((BLKN, 1), lambda i: (i, 0)),
        ],
        out_specs=[pl.BlockSpec((3 * F + 1, NG), lambda i: (0, 0))],
        out_shape=[jax.ShapeDtypeStruct((3 * F + 1, NG), jnp.float32)],
    )(h, a0, a1, batch2)


def _head_kernel(p1, p2, p3, w1t, b1r, w2t, b2r, w3t, b3r, *, num_out):
    def body(p1_ref, p2_ref, p3_ref, w1_ref, b1_ref, w2_ref, b2_ref,
             w3_ref, b3_ref, ot_ref, gt_ref):
        def finish(pref):
            pv = pref[...]
            mean = pv[2 * F : 3 * F] / jnp.maximum(pv[3 * F : 3 * F + 1], 1.0)
            return jnp.concatenate([pv[0 : 2 * F], mean], axis=0)

        g = finish(p1_ref) + finish(p2_ref) + finish(p3_ref)
        gt_ref[...] = g
        o = jnp.dot(w1_ref[...], g, preferred_element_type=jnp.float32)
        o = jnp.maximum(o + b1_ref[...], 0.0)
        o = jnp.dot(w2_ref[...], o, preferred_element_type=jnp.float32)
        o = jnp.maximum(o + b2_ref[...], 0.0)
        o = jnp.dot(w3_ref[...], o, preferred_element_type=jnp.float32)
        ot_ref[...] = o + b3_ref[...]

    return pl.pallas_call(
        body,
        out_shape=[
            jax.ShapeDtypeStruct((num_out, NG), jnp.float32),
            jax.ShapeDtypeStruct((3 * F, NG), jnp.float32),
        ],
    )(p1, p2, p3, w1t, b1r, w2t, b2r, w3t, b3r)


def _pack_layer(Wf, bf, Ws, bs):
    wd = jnp.concatenate([Wf[0:F], Ws[0:F]], axis=1)
    wsrc = jnp.concatenate([Wf[F : 2 * F], Ws[F : 2 * F]], axis=1)
    wea = jnp.concatenate([Wf[2 * F :], Ws[2 * F :]], axis=1)
    bia = jnp.concatenate([bf, bs])[None, :]
    return jnp.concatenate([wd, wsrc], axis=1), wea, bia


def kernel(x, edge_index, edge_attr, batch,
           Wf1, bf1, Ws1, bs1, Wf2, bf2, Ws2, bs2, Wf3, bf3, Ws3, bs3,
           W1, b1, W2, b2, W3, b3):
    n = x.shape[0]
    e = edge_index.shape[1]
    num_out = W3.shape[1]

    rpt = n // NS + 6
    nacc = NS * rpt
    nw = NC * NS
    cpw = -(-e // (nw * CH))
    cpw = cpw + (cpw % 2)
    epw = cpw * CH
    ep = nw * epw

    pad = ep - e
    padi = jnp.arange(pad, dtype=jnp.int32)
    dstp = jnp.concatenate([edge_index[1], n + (padi % (nacc - n))])
    srcp = jnp.concatenate([edge_index[0], (padi * 37) % n])
    dscat = jnp.stack(
        [dstp.reshape(-1, CH), srcp.reshape(-1, CH)], axis=1).reshape(-1, CH)
    zrows = jnp.zeros((rpt, MW), jnp.float32)
    batch2 = batch.reshape(n, 1)

    wc1, wea1, bia1 = _pack_layer(Wf1, bf1, Ws1, bs1)
    wc2, wea2, bia2 = _pack_layer(Wf2, bf2, Ws2, bs2)
    wc3, wea3, bia3 = _pack_layer(Wf3, bf3, Ws3, bs3)
    wea_all = jnp.concatenate([wea1, wea2, wea3], axis=1)
    bia_all = jnp.concatenate([bia1, bia2, bia3], axis=1)

    (eaw,) = _ea_kernel(edge_attr, wea_all, bia_all, ep=ep)
    wnext = (None, wc2, wc3)

    pd, ps = _proj_kernel(x, wc1, n=n, nacc=nacc)
    h = x
    pools = []
    for l in range(3):
        accf = _sc_edge_stage(pd, ps, eaw, dscat, zrows,
                              nacc=nacc, rpt=rpt, cpw=cpw, loff=32 * l)
        a0 = accf[:n, :F]
        a1 = accf[nacc : nacc + n, :F]
        if l < 2:
            h, pd, ps, pool = _upd_kernel(h, a0, a1, batch2, wnext[l + 1],
                                          n=n, nacc=nacc)
        else:
            (pool,) = _upd_last_kernel(h, a0, a1, batch2, n=n)
        pools.append(pool)

    ot, gt = _head_kernel(pools[0], pools[1], pools[2],
                          W1.T, b1.reshape(-1, 1),
                          W2.T, b2.reshape(-1, 1),
                          W3.T, b3.reshape(-1, 1), num_out=num_out)
    o = ot.T
    encoding = gt.T
    return (o, lax.stop_gradient(encoding))
